# baseline (speedup 1.0000x reference)
import functools

import jax
import jax.numpy as jnp
from jax import lax
from jax.experimental import pallas as pl
from jax.experimental.pallas import tpu as pltpu
from jax.experimental.pallas import tpu_sc as plsc

B, VIS, D, TOTAL = 512, 64, 96, 256
ROWS_OUT = TOTAL + 1
IMG_ROWS = VIS + 1
L = 16
DGRP = 8
BCH = 128
MASK_ROW = IMG_ROWS
NITEMS = 3
CHUNK = 16
NCH = 8


def _sc_revert(img_t, idx, mt):
    mesh = plsc.VectorSubcoreMesh(core_axis_name="c", subcore_axis_name="s")

    @functools.partial(
        pl.kernel,
        out_type=jax.ShapeDtypeStruct((ROWS_OUT, D, B), jnp.float32),
        mesh=mesh,
        compiler_params=pltpu.CompilerParams(needs_layout_passes=False),
        scratch_types=[
            pltpu.VMEM((IMG_ROWS + 1, DGRP, BCH), jnp.float32),
            pltpu.VMEM((BCH, BCH), jnp.int32),
            pltpu.VMEM((2, CHUNK, DGRP, BCH), jnp.float32),
            pltpu.VMEM((1, D), jnp.float32),
            pltpu.SemaphoreType.DMA,
            pltpu.SemaphoreType.DMA((2,)),
            pltpu.SemaphoreType.DMA,
        ],
    )
    def k(img_hbm, idx_hbm, mt_hbm, out_hbm, stage, idxv, outb, mtv,
          gsem, ssem, msem):
        wid = lax.axis_index("s") * 2 + lax.axis_index("c")
        lanes = lax.iota(jnp.int32, L)

        pltpu.async_copy(mt_hbm, mtv, msem).wait()

        def item_body(i, carry):
            item = wid * NITEMS + i
            dg = item // 8
            rem = item - dg * 8
            d0 = pl.multiple_of(dg * DGRP, DGRP)
            b0 = pl.multiple_of((rem // 2) * BCH, BCH)
            th = rem - (rem // 2) * 2
            to0 = th * (NCH * CHUNK)

            stage_cp = pltpu.async_copy(
                img_hbm.at[:, pl.ds(d0, DGRP), pl.ds(b0, BCH)],
                stage.at[pl.ds(0, IMG_ROWS)],
                gsem,
            )
            idx_cp = pltpu.async_copy(
                idx_hbm.at[pl.ds(b0, BCH), pl.ds(to0, BCH)],
                idxv,
                msem,
            )

            for dloc in range(DGRP):
                md = plsc.load_gather(
                    mtv, [jnp.zeros((L,), jnp.int32),
                          jnp.full((L,), d0 + dloc, jnp.int32)])
                for g in range(BCH // L):
                    stage[MASK_ROW, dloc, pl.ds(g * L, L)] = md

            stage_cp.wait()
            idx_cp.wait()

            @pl.when(th == 0)
            def _():
                pltpu.async_copy(
                    stage.at[pl.ds(0, 1)],
                    out_hbm.at[pl.ds(0, 1), pl.ds(d0, DGRP), pl.ds(b0, BCH)],
                    msem,
                ).wait()

            def drain_cp(c, slot):
                return pltpu.make_async_copy(
                    outb.at[slot],
                    out_hbm.at[pl.ds(to0 + c * CHUNK + 1, CHUNK),
                               pl.ds(d0, DGRP), pl.ds(b0, BCH)],
                    ssem.at[slot],
                )

            bls = [g * L + lanes for g in range(BCH // L)]
            dsplats = [jnp.full((L,), dd, jnp.int32) for dd in range(DGRP)]

            def chunk_body(c, carry):
                slot = lax.rem(c, 2)

                @pl.when(c >= 2)
                def _():
                    drain_cp(c - 2, slot).wait()

                @plsc.parallel_loop(0, CHUNK, unroll=8)
                def row(r):
                    tl = jnp.full((L,), c * CHUNK + r, jnp.int32)
                    for g in range(BCH // L):
                        j = plsc.load_gather(idxv, [bls[g], tl])
                        srcs = jnp.minimum(j + 1, MASK_ROW)
                        for dloc in range(DGRP):
                            v = plsc.load_gather(
                                stage, [srcs, dsplats[dloc], bls[g]])
                            outb[slot, r, dloc, pl.ds(g * L, L)] = v

                drain_cp(c, slot).start()
                return carry

            lax.fori_loop(0, NCH, chunk_body, 0)
            drain_cp(NCH - 2, 0).wait()
            drain_cp(NCH - 1, 1).wait()
            return carry

        lax.fori_loop(0, NITEMS, item_body, 0)

    return k(img_t, idx, mt)


def kernel(img, img_revert_idx, mask_token):
    img_t = jnp.transpose(img, (1, 2, 0))
    out_t = _sc_revert(img_t, img_revert_idx, mask_token)
    return jnp.transpose(out_t, (2, 0, 1))


# speedup vs baseline: 1.1347x; 1.1347x over previous
import functools

import jax
import jax.numpy as jnp
from jax import lax
from jax.experimental import pallas as pl
from jax.experimental.pallas import tpu as pltpu
from jax.experimental.pallas import tpu_sc as plsc

B, VIS, D, TOTAL = 512, 64, 96, 256
ROWS_OUT = TOTAL + 1
IMG_ROWS = VIS + 1
L = 16
DGRP = 8
BCH = 128
MASK_ROW = IMG_ROWS
NITEMS = 3
CHUNK = 16
NCH = 8


def _sc_revert(img_t, idx, mt):
    mesh = plsc.VectorSubcoreMesh(core_axis_name="c", subcore_axis_name="s")

    @functools.partial(
        pl.kernel,
        out_type=jax.ShapeDtypeStruct((ROWS_OUT, D, B), jnp.float32),
        mesh=mesh,
        compiler_params=pltpu.CompilerParams(needs_layout_passes=False),
        scratch_types=[
            pltpu.VMEM((IMG_ROWS + 1, DGRP, BCH), jnp.float32),
            pltpu.VMEM((BCH, BCH), jnp.int32),
            pltpu.VMEM((2, CHUNK, DGRP, BCH), jnp.float32),
            pltpu.VMEM((1, D), jnp.float32),
            pltpu.SemaphoreType.DMA,
            pltpu.SemaphoreType.DMA((2,)),
            pltpu.SemaphoreType.DMA,
        ],
    )
    def k(img_hbm, idx_hbm, mt_hbm, out_hbm, stage, idxv, outb, mtv,
          gsem, ssem, msem):
        wid = lax.axis_index("s") * 2 + lax.axis_index("c")
        lanes = lax.iota(jnp.int32, L)

        pltpu.async_copy(mt_hbm, mtv, msem).wait()

        def item_body(i, carry):
            item = wid * NITEMS + i
            dg = item // 8
            rem = item - dg * 8
            d0 = pl.multiple_of(dg * DGRP, DGRP)
            b0 = pl.multiple_of((rem // 2) * BCH, BCH)
            th = rem - (rem // 2) * 2
            to0 = th * (NCH * CHUNK)

            stage_cp = pltpu.async_copy(
                img_hbm.at[:, pl.ds(d0, DGRP), pl.ds(b0, BCH)],
                stage.at[pl.ds(0, IMG_ROWS)],
                gsem,
            )
            idx_cp = pltpu.async_copy(
                idx_hbm.at[pl.ds(b0, BCH), pl.ds(to0, BCH)],
                idxv,
                msem,
            )

            for dloc in range(DGRP):
                md = plsc.load_gather(
                    mtv, [jnp.zeros((L,), jnp.int32),
                          jnp.full((L,), d0 + dloc, jnp.int32)])
                for g in range(BCH // L):
                    stage[MASK_ROW, dloc, pl.ds(g * L, L)] = md

            stage_cp.wait()
            idx_cp.wait()

            @pl.when(th == 0)
            def _():
                pltpu.async_copy(
                    stage.at[pl.ds(0, 1)],
                    out_hbm.at[pl.ds(0, 1), pl.ds(d0, DGRP), pl.ds(b0, BCH)],
                    msem,
                ).wait()

            def drain_cp(c, slot):
                return pltpu.make_async_copy(
                    outb.at[slot],
                    out_hbm.at[pl.ds(to0 + c * CHUNK + 1, CHUNK),
                               pl.ds(d0, DGRP), pl.ds(b0, BCH)],
                    ssem.at[slot],
                )

            bls = [g * L + lanes for g in range(BCH // L)]
            dsplats = [jnp.full((L,), dd, jnp.int32) for dd in range(DGRP)]

            def chunk_body(c, carry):
                slot = lax.rem(c, 2)

                @pl.when(c >= 2)
                def _():
                    drain_cp(c - 2, slot).wait()

                @plsc.parallel_loop(0, CHUNK, unroll=4)
                def row(r):
                    tl = jnp.full((L,), c * CHUNK + r, jnp.int32)
                    for g in range(BCH // L):
                        j = plsc.load_gather(idxv, [bls[g], tl])
                        srcs = jnp.minimum(j + 1, MASK_ROW)
                        for dloc in range(DGRP):
                            v = plsc.load_gather(
                                stage, [srcs, dsplats[dloc], bls[g]])
                            outb[slot, r, dloc, pl.ds(g * L, L)] = v

                drain_cp(c, slot).start()
                return carry

            lax.fori_loop(0, NCH, chunk_body, 0)
            drain_cp(NCH - 2, 0).wait()
            drain_cp(NCH - 1, 1).wait()
            return carry

        lax.fori_loop(0, NITEMS, item_body, 0)

    return k(img_t, idx, mt)


def kernel(img, img_revert_idx, mask_token):
    img_t = jnp.transpose(img, (1, 2, 0))
    out_t = _sc_revert(img_t, img_revert_idx, mask_token)
    return jnp.transpose(out_t, (2, 0, 1))


# speedup vs baseline: 1.1952x; 1.0533x over previous
import functools

import jax
import jax.numpy as jnp
from jax import lax
from jax.experimental import pallas as pl
from jax.experimental.pallas import tpu as pltpu
from jax.experimental.pallas import tpu_sc as plsc

B, VIS, D, TOTAL = 512, 64, 96, 256
ROWS_OUT = TOTAL + 1
IMG_ROWS = VIS + 1
L = 16
DGRP = 8
BCH = 128
MASK_ROW = IMG_ROWS
NITEMS = 3
CHUNK = 16
NCH = 8


def _sc_revert(img_t, idx, mt):
    mesh = plsc.VectorSubcoreMesh(core_axis_name="c", subcore_axis_name="s")

    @functools.partial(
        pl.kernel,
        out_type=jax.ShapeDtypeStruct((ROWS_OUT, D, B), jnp.float32),
        mesh=mesh,
        compiler_params=pltpu.CompilerParams(needs_layout_passes=False),
        scratch_types=[
            pltpu.VMEM((IMG_ROWS + 1, DGRP, BCH), jnp.float32),
            pltpu.VMEM((BCH, BCH), jnp.int32),
            pltpu.VMEM((2, CHUNK, DGRP, BCH), jnp.float32),
            pltpu.VMEM((1, D), jnp.float32),
            pltpu.SemaphoreType.DMA,
            pltpu.SemaphoreType.DMA((2,)),
            pltpu.SemaphoreType.DMA,
            pltpu.SemaphoreType.DMA,
        ],
    )
    def k(img_hbm, idx_hbm, mt_hbm, out_hbm, stage, idxv, outb, mtv,
          gsem, ssem, msem, isem):
        wid = lax.axis_index("s") * 2 + lax.axis_index("c")
        lanes = lax.iota(jnp.int32, L)

        pltpu.async_copy(mt_hbm, mtv, msem).wait()

        def params_for(i):
            item = wid * NITEMS + i
            dg = item // 8
            rem = item - dg * 8
            d0 = pl.multiple_of(dg * DGRP, DGRP)
            b0 = pl.multiple_of((rem // 2) * BCH, BCH)
            th = rem - (rem // 2) * 2
            to0 = th * (NCH * CHUNK)
            return d0, b0, th, to0

        def stage_cp_for(i):
            d0, b0, _, _ = params_for(i)
            return pltpu.make_async_copy(
                img_hbm.at[:, pl.ds(d0, DGRP), pl.ds(b0, BCH)],
                stage.at[pl.ds(0, IMG_ROWS)],
                gsem,
            )

        def idx_cp_for(i):
            d0, b0, _, to0 = params_for(i)
            return pltpu.make_async_copy(
                idx_hbm.at[pl.ds(b0, BCH), pl.ds(to0, BCH)],
                idxv,
                isem,
            )

        stage_cp_for(0).start()
        idx_cp_for(0).start()

        def item_body(i, carry):
            d0, b0, th, to0 = params_for(i)

            for dloc in range(DGRP):
                md = plsc.load_gather(
                    mtv, [jnp.zeros((L,), jnp.int32),
                          jnp.full((L,), d0 + dloc, jnp.int32)])
                for g in range(BCH // L):
                    stage[MASK_ROW, dloc, pl.ds(g * L, L)] = md

            stage_cp_for(i).wait()
            idx_cp_for(i).wait()

            @pl.when(th == 0)
            def _():
                pltpu.async_copy(
                    stage.at[pl.ds(0, 1)],
                    out_hbm.at[pl.ds(0, 1), pl.ds(d0, DGRP), pl.ds(b0, BCH)],
                    msem,
                ).wait()

            def drain_cp(c, slot):
                return pltpu.make_async_copy(
                    outb.at[slot],
                    out_hbm.at[pl.ds(to0 + c * CHUNK + 1, CHUNK),
                               pl.ds(d0, DGRP), pl.ds(b0, BCH)],
                    ssem.at[slot],
                )

            def chunk_body(c, carry):
                slot = lax.rem(c, 2)

                @pl.when(c >= 2)
                def _():
                    drain_cp(c - 2, slot).wait()

                @plsc.parallel_loop(0, CHUNK, unroll=4)
                def row(r):
                    tl = c * CHUNK + r
                    for g in range(BCH // L):
                        bl = g * L + lanes
                        j = plsc.load_gather(
                            idxv, [bl, jnp.full((L,), tl, jnp.int32)])
                        srcs = jnp.where(j < VIS, j + 1, MASK_ROW)
                        for dloc in range(DGRP):
                            v = plsc.load_gather(
                                stage,
                                [srcs, jnp.full((L,), dloc, jnp.int32), bl])
                            outb[slot, r, dloc, pl.ds(g * L, L)] = v

                drain_cp(c, slot).start()
                return carry

            lax.fori_loop(0, NCH, chunk_body, 0)

            @pl.when(i + 1 < NITEMS)
            def _():
                stage_cp_for(i + 1).start()
                idx_cp_for(i + 1).start()

            drain_cp(NCH - 2, 0).wait()
            drain_cp(NCH - 1, 1).wait()
            return carry

        lax.fori_loop(0, NITEMS, item_body, 0)

    return k(img_t, idx, mt)


def kernel(img, img_revert_idx, mask_token):
    img_t = jnp.transpose(img, (1, 2, 0))
    out_t = _sc_revert(img_t, img_revert_idx, mask_token)
    return jnp.transpose(out_t, (2, 0, 1))
